# baseline (speedup 1.0000x reference)
import functools

import jax
import jax.numpy as jnp
from jax.experimental import pallas as pl
from jax.experimental.pallas import tpu as pltpu

_OFFSET = 16384
_BC = 262144


def _body(x_ref, ox_ref):
    ox_ref[...] = x_ref[...] + x_ref.dtype.type(_OFFSET)


@functools.lru_cache(maxsize=None)
def _make_call(nrows: int, nnz: int, idx_dtype: str):
    idt = jnp.dtype(idx_dtype)
    grid = -(-nnz // _BC)
    return pl.pallas_call(
        _body,
        grid=(grid,),
        in_specs=[pl.BlockSpec((nrows, _BC), lambda i: (0, i))],
        out_specs=pl.BlockSpec((nrows, _BC), lambda i: (0, i)),
        out_shape=jax.ShapeDtypeStruct((nrows, nnz), idt),
        compiler_params=pltpu.CompilerParams(
            dimension_semantics=("arbitrary",),
        ),
    )


def kernel(indices, values):
    nnz, ncols = indices.shape
    xt = indices.T
    yt = _make_call(ncols, nnz, str(indices.dtype))(xt)
    return (yt.T, values)


# speedup vs baseline: 1.1129x; 1.1129x over previous
import functools

import jax
import jax.numpy as jnp
from jax.experimental import pallas as pl
from jax.experimental.pallas import tpu as pltpu

_OFFSET = 16384
_BC = 897024


def _body(x_ref, ox_ref):
    ox_ref[...] = x_ref[...] + x_ref.dtype.type(_OFFSET)


@functools.lru_cache(maxsize=None)
def _make_call(nrows: int, nnz: int, idx_dtype: str):
    idt = jnp.dtype(idx_dtype)
    grid = -(-nnz // _BC)
    return pl.pallas_call(
        _body,
        grid=(grid,),
        in_specs=[pl.BlockSpec((nrows, _BC), lambda i: (0, i))],
        out_specs=pl.BlockSpec((nrows, _BC), lambda i: (0, i)),
        out_shape=jax.ShapeDtypeStruct((nrows, nnz), idt),
        compiler_params=pltpu.CompilerParams(
            dimension_semantics=("arbitrary",),
        ),
    )


def kernel(indices, values):
    nnz, ncols = indices.shape
    xt = indices.T
    yt = _make_call(ncols, nnz, str(indices.dtype))(xt)
    return (yt.T, values)


# speedup vs baseline: 1.1465x; 1.0302x over previous
import functools

import jax
import jax.numpy as jnp
from jax.experimental import pallas as pl
from jax.experimental.pallas import tpu as pltpu

_OFFSET = 16384
_BC = 1048576


def _body(x_ref, ox_ref):
    ox_ref[...] = x_ref[...] + x_ref.dtype.type(_OFFSET)


@functools.lru_cache(maxsize=None)
def _make_call(nrows: int, nnz: int, idx_dtype: str):
    idt = jnp.dtype(idx_dtype)
    grid = -(-nnz // _BC)
    return pl.pallas_call(
        _body,
        grid=(grid,),
        in_specs=[pl.BlockSpec((nrows, _BC), lambda i: (0, i))],
        out_specs=pl.BlockSpec((nrows, _BC), lambda i: (0, i)),
        out_shape=jax.ShapeDtypeStruct((nrows, nnz), idt),
        compiler_params=pltpu.CompilerParams(
            dimension_semantics=("arbitrary",),
        ),
    )


def kernel(indices, values):
    nnz, ncols = indices.shape
    xt = indices.T
    yt = _make_call(ncols, nnz, str(indices.dtype))(xt)
    return (yt.T, values)
